# baseline (speedup 1.0000x reference)
import functools
import math

import jax
import jax.numpy as jnp
from jax import lax
from jax.experimental import pallas as pl
from jax.experimental.pallas import tpu as pltpu
from jax.experimental.pallas import tpu_sc as plsc

_NC, _NS = 2, 16
_CH = 128
_L = 16

_LOG2PI = math.log(2.0 * math.pi)
_C1 = math.log(1.0 / 5.0 - 1.0 / 20.0) - 0.5 * _LOG2PI
_C2 = math.log(4.0 / 5.0 - 1.0 / 20.0) - 0.5 * _LOG2PI
_C3 = math.log(1.0 / 10.0) - 0.5 * _LOG2PI


def _sc_gather_e2(user_mu, user_logvar, idx2, z):
    V, D = user_mu.shape
    B = z.shape[0]
    nw = _NC * _NS
    n_ch = idx2.shape[0] // nw
    b_per_w = n_ch * _CH
    mesh = plsc.VectorSubcoreMesh(core_axis_name="c", subcore_axis_name="s")

    @functools.partial(
        pl.kernel,
        mesh=mesh,
        out_type=jax.ShapeDtypeStruct((B, D), jnp.float32),
        scratch_types=[
            pltpu.VMEM((n_ch, _CH), jnp.int32),
            pltpu.VMEM((2, _CH, D), jnp.float32),
            pltpu.VMEM((2, _CH, D), jnp.float32),
            pltpu.VMEM((2, _CH, D), jnp.float32),
            pltpu.SemaphoreType.DMA,
            pltpu.SemaphoreType.DMA,
            pltpu.SemaphoreType.DMA,
            pltpu.SemaphoreType.DMA,
        ],
    )
    def k(mu_hbm, lv_hbm, z_hbm, idx_hbm, e2_out, idx_v, mu_v, lv_v, z_v,
          in_sem0, in_sem1, out_sem0, out_sem1):
        wid = lax.axis_index("s") * _NC + lax.axis_index("c")
        base = wid * b_per_w
        in_sems = (in_sem0, in_sem1)
        out_sems = (out_sem0, out_sem1)
        pltpu.sync_copy(idx_hbm.at[pl.ds(wid * n_ch, n_ch)], idx_v)

        def fire(ch):
            p = ch % 2
            rows = pl.ds(base + ch * _CH, _CH)
            return (
                pltpu.async_copy(mu_hbm.at[idx_v.at[ch]], mu_v.at[p], in_sems[p]),
                pltpu.async_copy(lv_hbm.at[idx_v.at[ch]], lv_v.at[p], in_sems[p]),
                pltpu.async_copy(z_hbm.at[rows], z_v.at[p], in_sems[p]),
            )

        def compute(ch):
            p = ch % 2

            def row_body(r, _):
                for c in range(D // _L):
                    cs = pl.ds(c * _L, _L)
                    zv = z_v[p, r, cs]
                    mu = mu_v[p, r, cs]
                    lv = lv_v[p, r, cs]
                    t = zv - mu
                    acc = lv + t * t * jnp.exp(-lv)
                    z_v[p, r, cs] = jnp.exp(_C2 - 0.5 * acc)
                return _

            lax.fori_loop(0, _CH, row_body, None)

        in_cps = {0: fire(0)}
        out_cps = {}
        for ch in range(n_ch):
            p = ch % 2
            if ch >= 1:
                out_cps.pop(ch - 1).wait()
            if ch + 1 < n_ch:
                in_cps[ch + 1] = fire(ch + 1)
            for cp in in_cps.pop(ch):
                cp.wait()
            compute(ch)
            rows = pl.ds(base + ch * _CH, _CH)
            out_cps[ch] = pltpu.async_copy(z_v.at[p], e2_out.at[rows], out_sems[p])
        out_cps.pop(n_ch - 1).wait()

    return k(user_mu, user_logvar, z, idx2)


def _tc_math(z, e2, mu_p, lv_p, lv_u):
    B, D = z.shape
    blk = 2048

    def body(z_ref, e2_ref, mup_ref, lvp_ref, lvu_ref, o_ref):
        mup = mup_ref[...]
        lvp = lvp_ref[...]
        lvu = lvu_ref[...]
        a1 = -0.5 * jnp.exp(-lvp)
        b1 = _C1 - 0.5 * lvp
        a3 = -0.5 * jnp.exp(-lvu)
        b3 = _C3 - 0.5 * lvu
        zp2 = (z_ref[...] - mup) ** 2
        e1 = jnp.exp(a1 * zp2 + b1)
        e3 = jnp.exp(a3 * zp2 + b3)
        o_ref[...] = jnp.log(e1 + e2_ref[...] + e3)

    bs = pl.BlockSpec((blk, D), lambda i: (i, 0))
    ps = pl.BlockSpec((1, D), lambda i: (0, 0))
    return pl.pallas_call(
        body,
        grid=(B // blk,),
        in_specs=[bs, bs, ps, ps, ps],
        out_specs=bs,
        out_shape=jax.ShapeDtypeStruct((B, D), jnp.float32),
    )(z, e2, mu_p, lv_p, lv_u)


def kernel(z, idx, mu_prior, logvar_prior, logvar_uniform_prior, user_mu, user_logvar):
    B, D = z.shape
    idx2 = idx.astype(jnp.int32).reshape(-1, _CH)
    e2 = _sc_gather_e2(user_mu, user_logvar, idx2, z)
    return _tc_math(
        z,
        e2,
        mu_prior.reshape(1, D),
        logvar_prior.reshape(1, D),
        logvar_uniform_prior.reshape(1, D),
    )


# speedup vs baseline: 1.0441x; 1.0441x over previous
import functools
import math

import jax
import jax.numpy as jnp
from jax import lax
from jax.experimental import pallas as pl
from jax.experimental.pallas import tpu as pltpu
from jax.experimental.pallas import tpu_sc as plsc

_NC, _NS = 2, 16
_CH = 128
_L = 16

_LOG2PI = math.log(2.0 * math.pi)
_C1 = math.log(1.0 / 5.0 - 1.0 / 20.0) - 0.5 * _LOG2PI
_C2 = math.log(4.0 / 5.0 - 1.0 / 20.0) - 0.5 * _LOG2PI
_C3 = math.log(1.0 / 10.0) - 0.5 * _LOG2PI


def _sc_gather_e2(user_mu, user_logvar, idx2, z):
    V, D = user_mu.shape
    B = z.shape[0]
    nw = _NC * _NS
    n_ch = idx2.shape[0] // nw
    b_per_w = n_ch * _CH
    mesh = plsc.VectorSubcoreMesh(core_axis_name="c", subcore_axis_name="s")

    @functools.partial(
        pl.kernel,
        mesh=mesh,
        out_type=jax.ShapeDtypeStruct((B, D), jnp.float32),
        scratch_types=[
            pltpu.VMEM((n_ch, _CH), jnp.int32),
            pltpu.VMEM((2, _CH, D), jnp.float32),
            pltpu.VMEM((2, _CH, D), jnp.float32),
            pltpu.VMEM((2, _CH, D), jnp.float32),
            pltpu.SemaphoreType.DMA,
            pltpu.SemaphoreType.DMA,
            pltpu.SemaphoreType.DMA,
            pltpu.SemaphoreType.DMA,
        ],
    )
    def k(mu_hbm, lv_hbm, z_hbm, idx_hbm, e2_out, idx_v, mu_v, lv_v, z_v,
          in_sem0, in_sem1, out_sem0, out_sem1):
        wid = lax.axis_index("s") * _NC + lax.axis_index("c")
        base = wid * b_per_w
        in_sems = (in_sem0, in_sem1)
        out_sems = (out_sem0, out_sem1)
        pltpu.sync_copy(idx_hbm.at[pl.ds(wid * n_ch, n_ch)], idx_v)

        def fire(ch):
            p = ch % 2
            rows = pl.ds(base + ch * _CH, _CH)
            return (
                pltpu.async_copy(mu_hbm.at[idx_v.at[ch]], mu_v.at[p], in_sems[p]),
                pltpu.async_copy(lv_hbm.at[idx_v.at[ch]], lv_v.at[p], in_sems[p]),
                pltpu.async_copy(z_hbm.at[rows], z_v.at[p], in_sems[p]),
            )

        def compute(ch):
            p = ch % 2

            def row_body(r, _):
                for c in range(D // _L):
                    cs = pl.ds(c * _L, _L)
                    zv = z_v[p, r, cs]
                    mu = mu_v[p, r, cs]
                    lv = lv_v[p, r, cs]
                    t = zv - mu
                    acc = lv + t * t * jnp.exp(-lv)
                    z_v[p, r, cs] = jnp.exp(_C2 - 0.5 * acc)
                return _

            lax.fori_loop(0, _CH, row_body, None)

        in_cps = {0: fire(0)}
        out_cps = {}
        for ch in range(n_ch):
            p = ch % 2
            for cp in in_cps.pop(ch):
                cp.wait()
            if ch >= 1:
                out_cps.pop(ch - 1).wait()
            if ch + 1 < n_ch:
                in_cps[ch + 1] = fire(ch + 1)
            compute(ch)
            rows = pl.ds(base + ch * _CH, _CH)
            out_cps[ch] = pltpu.async_copy(z_v.at[p], e2_out.at[rows], out_sems[p])
        out_cps.pop(n_ch - 1).wait()

    return k(user_mu, user_logvar, z, idx2)


def _tc_math(z, e2, mu_p, lv_p, lv_u):
    B, D = z.shape
    blk = 2048

    def body(z_ref, e2_ref, mup_ref, lvp_ref, lvu_ref, o_ref):
        mup = mup_ref[...]
        lvp = lvp_ref[...]
        lvu = lvu_ref[...]
        a1 = -0.5 * jnp.exp(-lvp)
        b1 = _C1 - 0.5 * lvp
        a3 = -0.5 * jnp.exp(-lvu)
        b3 = _C3 - 0.5 * lvu
        zp2 = (z_ref[...] - mup) ** 2
        e1 = jnp.exp(a1 * zp2 + b1)
        e3 = jnp.exp(a3 * zp2 + b3)
        o_ref[...] = jnp.log(e1 + e2_ref[...] + e3)

    bs = pl.BlockSpec((blk, D), lambda i: (i, 0))
    ps = pl.BlockSpec((1, D), lambda i: (0, 0))
    return pl.pallas_call(
        body,
        grid=(B // blk,),
        in_specs=[bs, bs, ps, ps, ps],
        out_specs=bs,
        out_shape=jax.ShapeDtypeStruct((B, D), jnp.float32),
    )(z, e2, mu_p, lv_p, lv_u)


def kernel(z, idx, mu_prior, logvar_prior, logvar_uniform_prior, user_mu, user_logvar):
    B, D = z.shape
    idx2 = idx.astype(jnp.int32).reshape(-1, _CH)
    e2 = _sc_gather_e2(user_mu, user_logvar, idx2, z)
    return _tc_math(
        z,
        e2,
        mu_prior.reshape(1, D),
        logvar_prior.reshape(1, D),
        logvar_uniform_prior.reshape(1, D),
    )
